# baseline (speedup 1.0000x reference)
import functools

import jax
import jax.numpy as jnp
from jax import lax
from jax.experimental import pallas as pl
from jax.experimental.pallas import tpu as pltpu
from jax.experimental.pallas import tpu_sc as plsc

_NUM_CLASSES = 19
_BH = 128
_NC, _NS, _L = 2, 16, 16
_NW = _NC * _NS
_CHUNK = 8192


def _argmax_body(x_ref, o_ref):
    x = x_ref[0]
    m = x[0]
    idx = jnp.zeros(m.shape, dtype=jnp.int32)
    for c in range(1, _NUM_CLASSES):
        v = x[c]
        gt = v > m
        m = jnp.where(gt, v, m)
        idx = jnp.where(gt, c, idx)
    o_ref[0] = idx


def _tc_argmax(soft_label):
    b, nc, h, wdim = soft_label.shape
    return pl.pallas_call(
        _argmax_body,
        grid=(b, h // _BH),
        in_specs=[pl.BlockSpec((1, nc, _BH, wdim), lambda i, j: (i, 0, j, 0))],
        out_specs=pl.BlockSpec((1, _BH, wdim), lambda i, j: (i, j, 0)),
        out_shape=jax.ShapeDtypeStruct((b, h, wdim), jnp.int32),
    )(soft_label)


def _make_sc_gather(n):
    per_w = n // _NW
    mesh = plsc.VectorSubcoreMesh(
        core_axis_name="c", subcore_axis_name="s",
        num_cores=_NC, num_subcores=_NS)

    @functools.partial(
        pl.kernel,
        mesh=mesh,
        compiler_params=pltpu.CompilerParams(needs_layout_passes=False),
        out_type=jax.ShapeDtypeStruct((n,), jnp.float32),
        scratch_types=[
            pltpu.VMEM((2 * _L,), jnp.float32),
            pltpu.VMEM((_CHUNK,), jnp.int32),
            pltpu.VMEM((_CHUNK,), jnp.float32),
        ],
    )
    def _sc_gather(tbl_hbm, idx_hbm, out_hbm, tbl_v, idx_v, out_v):
        wid = lax.axis_index("s") * _NC + lax.axis_index("c")
        base_w = wid * per_w
        pltpu.sync_copy(tbl_hbm, tbl_v)

        def chunk_body(ci, _):
            base = base_w + ci * _CHUNK
            pltpu.sync_copy(idx_hbm.at[pl.ds(base, _CHUNK)], idx_v)

            def vec_body(i, _):
                iv = idx_v[pl.ds(i * _L, _L)]
                out_v[pl.ds(i * _L, _L)] = plsc.load_gather(tbl_v, [iv])
                return 0

            lax.fori_loop(0, _CHUNK // _L, vec_body, 0, unroll=4)
            pltpu.sync_copy(out_v, out_hbm.at[pl.ds(base, _CHUNK)])
            return 0

        lax.fori_loop(0, per_w // _CHUNK, chunk_body, 0)

    return _sc_gather


@jax.jit
def kernel(soft_label, pixel_weights):
    b, nc, h, wdim = soft_label.shape
    n = b * h * wdim
    idx = _tc_argmax(soft_label)
    tbl = jnp.zeros((2 * _L,), jnp.float32).at[:nc].set(pixel_weights)
    out = _make_sc_gather(n)(tbl, idx.reshape(n))
    return out.reshape(b, h, wdim)


# speedup vs baseline: 1.2709x; 1.2709x over previous
import functools

import jax
import jax.numpy as jnp
from jax import lax
from jax.experimental import pallas as pl
from jax.experimental.pallas import tpu as pltpu
from jax.experimental.pallas import tpu_sc as plsc

_NUM_CLASSES = 19
_BH = 128
_NC, _NS, _L = 2, 16, 16
_NW = _NC * _NS
_CHUNK = 8192


def _argmax_body(x_ref, o_ref):
    x = x_ref[0]
    m = x[0]
    idx = jnp.zeros(m.shape, dtype=jnp.int32)
    for c in range(1, _NUM_CLASSES):
        v = x[c]
        gt = v > m
        m = jnp.where(gt, v, m)
        idx = jnp.where(gt, c, idx)
    o_ref[0] = idx


def _tc_argmax(soft_label):
    b, nc, h, wdim = soft_label.shape
    return pl.pallas_call(
        _argmax_body,
        grid=(b, h // _BH),
        in_specs=[pl.BlockSpec((1, nc, _BH, wdim), lambda i, j: (i, 0, j, 0))],
        out_specs=pl.BlockSpec((1, _BH, wdim), lambda i, j: (i, j, 0)),
        out_shape=jax.ShapeDtypeStruct((b, h, wdim), jnp.int32),
    )(soft_label)


def _make_sc_gather(n):
    per_w = n // _NW
    mesh = plsc.VectorSubcoreMesh(
        core_axis_name="c", subcore_axis_name="s",
        num_cores=_NC, num_subcores=_NS)

    @functools.partial(
        pl.kernel,
        mesh=mesh,
        compiler_params=pltpu.CompilerParams(needs_layout_passes=False),
        out_type=jax.ShapeDtypeStruct((n,), jnp.float32),
        scratch_types=[
            pltpu.VMEM((_NUM_CLASSES,), jnp.float32),
            pltpu.VMEM((_CHUNK,), jnp.int32),
            pltpu.VMEM((_CHUNK,), jnp.float32),
        ],
    )
    def _sc_gather(tbl_hbm, idx_hbm, out_hbm, tbl_v, idx_v, out_v):
        wid = lax.axis_index("s") * _NC + lax.axis_index("c")
        base_w = wid * per_w
        pltpu.sync_copy(tbl_hbm, tbl_v)

        def chunk_body(ci, _):
            base = base_w + ci * _CHUNK
            pltpu.sync_copy(idx_hbm.at[pl.ds(base, _CHUNK)], idx_v)

            @plsc.parallel_loop(0, _CHUNK // _L, unroll=8)
            def vec_body(i):
                iv = idx_v[pl.ds(i * _L, _L)]
                out_v[pl.ds(i * _L, _L)] = plsc.load_gather(tbl_v, [iv])

            pltpu.sync_copy(out_v, out_hbm.at[pl.ds(base, _CHUNK)])
            return 0

        lax.fori_loop(0, per_w // _CHUNK, chunk_body, 0)

    return _sc_gather


@jax.jit
def kernel(soft_label, pixel_weights):
    b, nc, h, wdim = soft_label.shape
    n = b * h * wdim
    idx = _tc_argmax(soft_label)
    out = _make_sc_gather(n)(pixel_weights, idx.reshape(n))
    return out.reshape(b, h, wdim)


# speedup vs baseline: 1.6258x; 1.2793x over previous
import functools

import jax
import jax.numpy as jnp
from jax import lax
from jax.experimental import pallas as pl
from jax.experimental.pallas import tpu as pltpu
from jax.experimental.pallas import tpu_sc as plsc

_NUM_CLASSES = 19
_BH = 128
_NC, _NS, _L = 2, 16, 16
_NW = _NC * _NS
_CROWS = 32


def _argmax_body(x_ref, o_ref):
    x = x_ref[0]
    m = x[0]
    idx = jnp.zeros(m.shape, dtype=jnp.int32)
    for c in range(1, _NUM_CLASSES):
        v = x[c]
        gt = v > m
        m = jnp.where(gt, v, m)
        idx = jnp.where(gt, c, idx)
    o_ref[0] = idx


def _tc_argmax(soft_label):
    b, nc, h, wdim = soft_label.shape
    return pl.pallas_call(
        _argmax_body,
        grid=(b, h // _BH),
        in_specs=[pl.BlockSpec((1, nc, _BH, wdim), lambda i, j: (i, 0, j, 0))],
        out_specs=pl.BlockSpec((1, _BH, wdim), lambda i, j: (i, j, 0)),
        out_shape=jax.ShapeDtypeStruct((b, h, wdim), jnp.int32),
    )(soft_label)


def _make_sc_gather(b, h, wdim):
    rows_per_w = (b * h) // _NW
    n_chunks = rows_per_w // _CROWS
    vecs = (_CROWS * wdim) // _L
    cols = wdim // _L
    mesh = plsc.VectorSubcoreMesh(
        core_axis_name="c", subcore_axis_name="s",
        num_cores=_NC, num_subcores=_NS)

    @functools.partial(
        pl.kernel,
        mesh=mesh,
        compiler_params=pltpu.CompilerParams(
            needs_layout_passes=False, use_tc_tiling_on_sc=True),
        out_type=jax.ShapeDtypeStruct((b, h, wdim), jnp.float32),
        scratch_types=[
            pltpu.VMEM((_NUM_CLASSES,), jnp.float32),
            pltpu.VMEM((_CROWS, wdim), jnp.int32),
            pltpu.VMEM((_CROWS, wdim), jnp.float32),
        ],
    )
    def _sc_gather(tbl_hbm, idx_hbm, out_hbm, tbl_v, idx_v, out_v):
        wid = lax.axis_index("s") * _NC + lax.axis_index("c")
        row_w = wid * rows_per_w
        rows_per_b = h

        def chunk_body(ci, _):
            row = row_w + ci * _CROWS
            bb = row // rows_per_b
            rr = row % rows_per_b
            pltpu.sync_copy(idx_hbm.at[bb, pl.ds(rr, _CROWS)], idx_v)

            @plsc.parallel_loop(0, vecs, unroll=8)
            def vec_body(i):
                r = i // cols
                c = (i % cols) * _L
                iv = idx_v[r, pl.ds(c, _L)]
                out_v[r, pl.ds(c, _L)] = plsc.load_gather(tbl_v, [iv])

            pltpu.sync_copy(out_v, out_hbm.at[bb, pl.ds(rr, _CROWS)])
            return 0

        pltpu.sync_copy(tbl_hbm, tbl_v)
        lax.fori_loop(0, n_chunks, chunk_body, 0)

    return _sc_gather


@jax.jit
def kernel(soft_label, pixel_weights):
    b, nc, h, wdim = soft_label.shape
    idx = _tc_argmax(soft_label)
    return _make_sc_gather(b, h, wdim)(pixel_weights, idx)
